# baseline (device time: 24530 ns/iter reference)
import jax
import jax.numpy as jnp
from jax import lax
from jax.experimental import pallas as pl
from jax.experimental.pallas import tpu as pltpu

N_Z = 4
N_Q = 4
N_S = 4
EPS = 1e-6


def kernel(partial, gamma):
    _, m, d = partial.shape
    part = jnp.reshape(partial, (m, d))
    ch = m // N_Z
    qd = d // N_Q
    sg = ch // N_S
    gamma2 = jnp.reshape(gamma, (1, d))

    def body(
        part_ref,
        gamma_ref,
        out_ref,
        zrecv_ref,
        qacc_ref,
        zsend_sems,
        zrecv_sems,
        qsend_sems,
        qrecv_sems,
    ):
        my_x = lax.axis_index("x")
        my_y = lax.axis_index("y")
        my_z = lax.axis_index("z")
        q = 2 * my_x + my_y

        barrier_sem = pltpu.get_barrier_semaphore()
        for dz in range(1, N_Z):
            pl.semaphore_signal(
                barrier_sem,
                inc=1,
                device_id=(my_x, my_y, (my_z + dz) % N_Z),
                device_id_type=pl.DeviceIdType.MESH,
            )
        xy_peers = (
            (1 - my_x, 1 - my_y, my_z),
            (1 - my_x, my_y, my_z),
            (my_x, 1 - my_y, my_z),
        )
        for nbr in xy_peers:
            pl.semaphore_signal(
                barrier_sem,
                inc=1,
                device_id=nbr,
                device_id_type=pl.DeviceIdType.MESH,
            )
        pl.semaphore_wait(barrier_sem, 6)

        zrdmas = [[None] * (N_Z - 1) for _ in range(N_S)]
        for s in range(N_S):
            for j in range(N_Z - 1):
                c = (my_z + j + 1) % N_Z
                rdma = pltpu.make_async_remote_copy(
                    src_ref=part_ref.at[
                        pl.ds(c * ch + s * sg, sg), pl.ds(q * qd, qd)
                    ],
                    dst_ref=zrecv_ref.at[s, j],
                    send_sem=zsend_sems.at[s, j],
                    recv_sem=zrecv_sems.at[s, j],
                    device_id=(my_x, my_y, c),
                    device_id_type=pl.DeviceIdType.MESH,
                )
                rdma.start()
                zrdmas[s][j] = rdma

        def norm_store(s):
            rows = pl.ds(s * sg, sg)
            sumsq = (
                jnp.sum(qacc_ref[0, rows, :] ** 2, axis=-1, keepdims=True)
                + jnp.sum(qacc_ref[1, rows, :] ** 2, axis=-1, keepdims=True)
                + jnp.sum(qacc_ref[2, rows, :] ** 2, axis=-1, keepdims=True)
                + jnp.sum(qacc_ref[3, rows, :] ** 2, axis=-1, keepdims=True)
            )
            inv = lax.rsqrt(sumsq / d + EPS)
            for j in range(N_Q):
                out_ref[rows, pl.ds(j * qd, qd)] = (
                    qacc_ref[j, rows, :] * inv * gamma_ref[:, pl.ds(j * qd, qd)]
                )

        qrdmas = [[None] * 3 for _ in range(N_S)]
        for s in range(N_S):
            for j in range(N_Z - 1):
                zrdmas[s][j].wait()
            qacc_ref[q, pl.ds(s * sg, sg), :] = (
                part_ref[pl.ds(my_z * ch + s * sg, sg), pl.ds(q * qd, qd)]
                + zrecv_ref[s, 0]
                + zrecv_ref[s, 1]
                + zrecv_ref[s, 2]
            )
            for j, nbr in enumerate(xy_peers):
                rdma = pltpu.make_async_remote_copy(
                    src_ref=qacc_ref.at[q, pl.ds(s * sg, sg)],
                    dst_ref=qacc_ref.at[q, pl.ds(s * sg, sg)],
                    send_sem=qsend_sems.at[s, j],
                    recv_sem=qrecv_sems.at[s, j],
                    device_id=nbr,
                    device_id_type=pl.DeviceIdType.MESH,
                )
                rdma.start()
                qrdmas[s][j] = rdma
            if s >= 1:
                for j in range(3):
                    qrdmas[s - 1][j].wait()
                norm_store(s - 1)
        for j in range(3):
            qrdmas[N_S - 1][j].wait()
        norm_store(N_S - 1)

    return pl.pallas_call(
        body,
        out_shape=jax.ShapeDtypeStruct((ch, d), jnp.float32),
        in_specs=[
            pl.BlockSpec(memory_space=pltpu.VMEM),
            pl.BlockSpec(memory_space=pltpu.VMEM),
        ],
        out_specs=pl.BlockSpec(memory_space=pltpu.VMEM),
        scratch_shapes=[
            pltpu.VMEM((N_S, N_Z - 1, sg, qd), jnp.float32),
            pltpu.VMEM((N_Q, ch, qd), jnp.float32),
            pltpu.SemaphoreType.DMA((N_S, N_Z - 1)),
            pltpu.SemaphoreType.DMA((N_S, N_Z - 1)),
            pltpu.SemaphoreType.DMA((N_S, 3)),
            pltpu.SemaphoreType.DMA((N_S, 3)),
        ],
        compiler_params=pltpu.CompilerParams(collective_id=0),
    )(part, gamma2)
